# baseline (speedup 1.0000x reference)
import jax
import jax.numpy as jnp
from jax import lax
from jax.experimental import pallas as pl
from jax.experimental.pallas import tpu as pltpu
from jax.experimental.pallas import tpu_sc as plsc

N = 10000
E = 320000
H = 128

NP = 10112
ROWS_PER_TILE = NP // 16
DUMMY_DST = 10008

EPAD = 327680
CH = 64
C0 = 320
C1 = 0

NSLOT = 5
ILAG = 3
GLAG = 2

BN = 400
NB = N // BN


def _mm_body(x_ref, w_ref, o_ref):
    xb = x_ref[...]
    for r in range(9):
        o_ref[r] = jnp.dot(xb, w_ref[r], preferred_element_type=jnp.float32)


def _layer1_matmul(x, wc):
    return pl.pallas_call(
        _mm_body,
        grid=(NB,),
        in_specs=[
            pl.BlockSpec((BN, H), lambda i: (i, 0)),
            pl.BlockSpec((9, H, H), lambda i: (0, 0, 0)),
        ],
        out_specs=pl.BlockSpec((9, BN, H), lambda i: (0, i, 0)),
        out_shape=jax.ShapeDtypeStruct((9, N, H), jnp.float32),
    )(x, wc)


def _mmc_body(a_ref, s_ref, b_ref, w_ref, o_ref, h_ref):
    hb = jnp.maximum(a_ref[0] + a_ref[1] + s_ref[0] + b_ref[...], 0.0)
    h_ref[...] = hb
    for r in range(9):
        o_ref[r] = jnp.dot(hb, w_ref[r], preferred_element_type=jnp.float32)


def _layer_combine_matmul(agg, hw_prev, b, wc):
    return pl.pallas_call(
        _mmc_body,
        grid=(NB,),
        in_specs=[
            pl.BlockSpec((2, BN, H), lambda i: (0, i, 0)),
            pl.BlockSpec((1, BN, H), lambda i: (8, i, 0)),
            pl.BlockSpec((1, H), lambda i: (0, 0)),
            pl.BlockSpec((9, H, H), lambda i: (0, 0, 0)),
        ],
        out_specs=[
            pl.BlockSpec((9, BN, H), lambda i: (0, i, 0)),
            pl.BlockSpec((BN, H), lambda i: (i, 0)),
        ],
        out_shape=[
            jax.ShapeDtypeStruct((9, N, H), jnp.float32),
            jax.ShapeDtypeStruct((N, H), jnp.float32),
        ],
    )(agg, hw_prev, b, wc)


def _final_body(h1_ref, h2_ref, a_ref, s_ref, b_ref, o_ref):
    h3 = jnp.maximum(a_ref[0] + a_ref[1] + s_ref[0] + b_ref[...], 0.0)
    row = jnp.concatenate(
        [
            jnp.sum(h1_ref[...], axis=0, keepdims=True),
            jnp.sum(h2_ref[...], axis=0, keepdims=True),
            jnp.sum(h3, axis=0, keepdims=True),
        ],
        axis=1,
    )
    i = pl.program_id(0)

    @pl.when(i == 0)
    def _():
        o_ref[...] = row

    @pl.when(i != 0)
    def _():
        o_ref[...] += row


def _final_pool(h1, h2, agg, hw_prev, b):
    return pl.pallas_call(
        _final_body,
        grid=(NB,),
        in_specs=[
            pl.BlockSpec((BN, H), lambda i: (i, 0)),
            pl.BlockSpec((BN, H), lambda i: (i, 0)),
            pl.BlockSpec((2, BN, H), lambda i: (0, i, 0)),
            pl.BlockSpec((1, BN, H), lambda i: (8, i, 0)),
            pl.BlockSpec((1, H), lambda i: (0, 0)),
        ],
        out_specs=pl.BlockSpec((1, 3 * H), lambda i: (0, 0)),
        out_shape=jax.ShapeDtypeStruct((1, 3 * H), jnp.float32),
    )(h1, h2, agg, hw_prev, b)


def _edge_kernel(hw_hbm, gidx_hbm, dst_hbm, zeros_hbm, out_hbm,
                 gidx_r, dst_r, rows_v, acc_sh, isem, jsem, gsem, ssem, zsem):
    cid = lax.axis_index("c")
    sid = lax.axis_index("s")

    zcp = pltpu.make_async_copy(
        zeros_hbm.at[pl.ds(sid * ROWS_PER_TILE, ROWS_PER_TILE)],
        acc_sh.at[pl.ds(sid * ROWS_PER_TILE, ROWS_PER_TILE)], zsem)
    zcp.start()

    def make_pipeline(nchunk, cbase):
        ng = nchunk // NSLOT

        def idx_start(slot, c):
            base = (cbase + c) * CH
            pltpu.make_async_copy(gidx_hbm.at[pl.ds(base, CH)],
                                  gidx_r.at[slot], isem.at[slot]).start()
            pltpu.make_async_copy(dst_hbm.at[pl.ds(base, CH)],
                                  dst_r.at[slot], jsem.at[slot]).start()

        def idx_wait(slot):
            pltpu.make_async_copy(gidx_hbm.at[pl.ds(0, CH)],
                                  gidx_r.at[slot], isem.at[slot]).wait()
            pltpu.make_async_copy(dst_hbm.at[pl.ds(0, CH)],
                                  dst_r.at[slot], jsem.at[slot]).wait()

        def gat_start(slot):
            pltpu.make_async_copy(hw_hbm.at[gidx_r.at[slot]],
                                  rows_v.at[slot], gsem.at[slot]).start()

        def gat_wait(slot):
            pltpu.make_async_copy(hw_hbm.at[gidx_r.at[slot]],
                                  rows_v.at[slot], gsem.at[slot]).wait()

        def sca_start(slot):
            pltpu.async_copy(rows_v.at[slot], acc_sh.at[dst_r.at[slot]],
                             ssem.at[slot], add=True)

        def sca_wait(slot):
            pltpu.make_async_copy(rows_v.at[slot], acc_sh.at[dst_r.at[slot]],
                                  ssem.at[slot]).wait()

        def steady(t, toff, first=False, last=False):
            if not first or t >= NSLOT - ILAG:
                sca_wait((t - (NSLOT - ILAG)) % NSLOT)
            if not last or t + ILAG < nchunk:
                idx_start((t + ILAG) % NSLOT, toff + ILAG)
            if not last or t + GLAG < nchunk:
                idx_wait((t + GLAG) % NSLOT)
                gat_start((t + GLAG) % NSLOT)
            gat_wait(t % NSLOT)
            sca_start(t % NSLOT)

        for c in range(ILAG):
            idx_start(c % NSLOT, c)
        for c in range(GLAG):
            idx_wait(c % NSLOT)
            gat_start(c % NSLOT)
        zcp.wait()
        plsc.subcore_barrier()

        for b in range(NSLOT):
            steady(b, b, first=True)

        def outer(G, carry):
            for b in range(NSLOT):
                steady(NSLOT + b, G * NSLOT + b)
            return carry

        lax.fori_loop(1, ng - 1, outer, 0)

        for b in range(NSLOT):
            steady((ng - 1) * NSLOT + b, (ng - 1) * NSLOT + b, last=True)

        for c in range(nchunk - (NSLOT - ILAG), nchunk):
            sca_wait(c % NSLOT)

    @pl.when(cid == 0)
    def _():
        make_pipeline(C0, sid * (C0 + C1))

    if C1 > 0:
        @pl.when(cid == 1)
        def _():
            make_pipeline(C1, sid * (C0 + C1) + C0)
    else:
        @pl.when(cid == 1)
        def _():
            zcp.wait()
            plsc.subcore_barrier()

    plsc.subcore_barrier()

    pltpu.sync_copy(acc_sh.at[pl.ds(sid * ROWS_PER_TILE, ROWS_PER_TILE)],
                    out_hbm.at[cid, pl.ds(sid * ROWS_PER_TILE, ROWS_PER_TILE)])


def _edge_aggregate(hw_flat, gidx, dst, zeros):
    mesh = plsc.VectorSubcoreMesh(core_axis_name="c", subcore_axis_name="s")
    return pl.kernel(
        _edge_kernel,
        mesh=mesh,
        out_type=jax.ShapeDtypeStruct((2, NP, H), jnp.float32),
        scratch_types=[
            pltpu.VMEM((NSLOT, CH), jnp.int32),
            pltpu.VMEM((NSLOT, CH), jnp.int32),
            pltpu.VMEM((NSLOT, CH, H), jnp.float32),
            pltpu.VMEM_SHARED((NP, H), jnp.float32),
            pltpu.SemaphoreType.DMA((NSLOT,)),
            pltpu.SemaphoreType.DMA((NSLOT,)),
            pltpu.SemaphoreType.DMA((NSLOT,)),
            pltpu.SemaphoreType.DMA((NSLOT,)),
            pltpu.SemaphoreType.DMA,
        ],
    )(hw_flat, gidx, dst, zeros)


def kernel(x, edge_index, edge_type, W1, Ws1, b1, W2, Ws2, b2, W3, Ws3, b3):
    src = edge_index[0]
    dst = edge_index[1]

    gidx = edge_type * N + src
    npad = EPAD - E
    gidx_p = jnp.concatenate([gidx, jnp.zeros((npad,), jnp.int32)])
    dst_p = jnp.concatenate([dst, jnp.full((npad,), DUMMY_DST, jnp.int32)])
    zeros = jnp.zeros((NP, H), jnp.float32)

    wc1 = jnp.concatenate([W1, Ws1[None]], axis=0)
    wc2 = jnp.concatenate([W2, Ws2[None]], axis=0)
    wc3 = jnp.concatenate([W3, Ws3[None]], axis=0)

    hw1 = _layer1_matmul(x, wc1)
    agg1 = _edge_aggregate(hw1.reshape(9 * N, H), gidx_p, dst_p, zeros)

    hw2, h1 = _layer_combine_matmul(agg1, hw1, b1.reshape(1, H), wc2)
    agg2 = _edge_aggregate(hw2.reshape(9 * N, H), gidx_p, dst_p, zeros)

    hw3, h2 = _layer_combine_matmul(agg2, hw2, b2.reshape(1, H), wc3)
    agg3 = _edge_aggregate(hw3.reshape(9 * N, H), gidx_p, dst_p, zeros)

    out = _final_pool(h1, h2, agg3, hw3, b3.reshape(1, H))
    return out.reshape(1, 1, 3 * H)


# speedup vs baseline: 1.1654x; 1.1654x over previous
import jax
import jax.numpy as jnp
from jax import lax
from jax.experimental import pallas as pl
from jax.experimental.pallas import tpu as pltpu
from jax.experimental.pallas import tpu_sc as plsc

N = 10000
E = 320000
H = 128

NP = 10112
ROWS_PER_TILE = NP // 16

EPAD = 327680
CH = 64
C0 = 160
C1 = 160

NSLOT = 5
ILAG = 3
GLAG = 2

BN = 400
NB = N // BN


def _mm_body(x_ref, w_ref, o_ref):
    xb = x_ref[...]
    for r in range(9):
        o_ref[r] = jnp.dot(xb, w_ref[r], preferred_element_type=jnp.float32)


def _layer1_matmul(x, wc):
    return pl.pallas_call(
        _mm_body,
        grid=(NB,),
        in_specs=[
            pl.BlockSpec((BN, H), lambda i: (i, 0)),
            pl.BlockSpec((9, H, H), lambda i: (0, 0, 0)),
        ],
        out_specs=pl.BlockSpec((9, BN, H), lambda i: (0, i, 0)),
        out_shape=jax.ShapeDtypeStruct((9, N, H), jnp.float32),
    )(x, wc)


def _mmc_body(a_ref, s_ref, b_ref, w_ref, o_ref, h_ref):
    hb = jnp.maximum(a_ref[0] + a_ref[1] + s_ref[0] + b_ref[...], 0.0)
    h_ref[...] = hb
    for r in range(9):
        o_ref[r] = jnp.dot(hb, w_ref[r], preferred_element_type=jnp.float32)


def _layer_combine_matmul(agg, hw_prev, b, wc):
    return pl.pallas_call(
        _mmc_body,
        grid=(NB,),
        in_specs=[
            pl.BlockSpec((2, BN, H), lambda i: (0, i, 0)),
            pl.BlockSpec((1, BN, H), lambda i: (8, i, 0)),
            pl.BlockSpec((1, H), lambda i: (0, 0)),
            pl.BlockSpec((9, H, H), lambda i: (0, 0, 0)),
        ],
        out_specs=[
            pl.BlockSpec((9, BN, H), lambda i: (0, i, 0)),
            pl.BlockSpec((BN, H), lambda i: (i, 0)),
        ],
        out_shape=[
            jax.ShapeDtypeStruct((9, N, H), jnp.float32),
            jax.ShapeDtypeStruct((N, H), jnp.float32),
        ],
    )(agg, hw_prev, b, wc)


def _final_body(h1_ref, h2_ref, a_ref, s_ref, b_ref, o_ref):
    h3 = jnp.maximum(a_ref[0] + a_ref[1] + s_ref[0] + b_ref[...], 0.0)
    row = jnp.concatenate(
        [
            jnp.sum(h1_ref[...], axis=0, keepdims=True),
            jnp.sum(h2_ref[...], axis=0, keepdims=True),
            jnp.sum(h3, axis=0, keepdims=True),
        ],
        axis=1,
    )
    i = pl.program_id(0)

    @pl.when(i == 0)
    def _():
        o_ref[...] = row

    @pl.when(i != 0)
    def _():
        o_ref[...] += row


def _final_pool(h1, h2, agg, hw_prev, b):
    return pl.pallas_call(
        _final_body,
        grid=(NB,),
        in_specs=[
            pl.BlockSpec((BN, H), lambda i: (i, 0)),
            pl.BlockSpec((BN, H), lambda i: (i, 0)),
            pl.BlockSpec((2, BN, H), lambda i: (0, i, 0)),
            pl.BlockSpec((1, BN, H), lambda i: (8, i, 0)),
            pl.BlockSpec((1, H), lambda i: (0, 0)),
        ],
        out_specs=pl.BlockSpec((1, 3 * H), lambda i: (0, 0)),
        out_shape=jax.ShapeDtypeStruct((1, 3 * H), jnp.float32),
    )(h1, h2, agg, hw_prev, b)


def _edge_kernel(hw_hbm, gidx_hbm, dst_hbm, zeros_hbm, out_hbm,
                 gidx_r, dst_r, rows_v, acc_sh, isem, jsem, gsem, ssem, zsem):
    cid = lax.axis_index("c")
    sid = lax.axis_index("s")

    zcp = pltpu.make_async_copy(
        zeros_hbm.at[pl.ds(sid * ROWS_PER_TILE, ROWS_PER_TILE)],
        acc_sh.at[pl.ds(sid * ROWS_PER_TILE, ROWS_PER_TILE)], zsem)
    zcp.start()

    def make_pipeline(nchunk, cbase):
        ng = nchunk // NSLOT

        def idx_start(slot, c):
            base = (cbase + c) * CH
            pltpu.make_async_copy(gidx_hbm.at[pl.ds(base, CH)],
                                  gidx_r.at[slot], isem.at[slot]).start()
            pltpu.make_async_copy(dst_hbm.at[pl.ds(base, CH)],
                                  dst_r.at[slot], jsem.at[slot]).start()

        def idx_wait(slot):
            pltpu.make_async_copy(gidx_hbm.at[pl.ds(0, CH)],
                                  gidx_r.at[slot], isem.at[slot]).wait()
            pltpu.make_async_copy(dst_hbm.at[pl.ds(0, CH)],
                                  dst_r.at[slot], jsem.at[slot]).wait()

        def gat_start(slot):
            pltpu.make_async_copy(hw_hbm.at[gidx_r.at[slot]],
                                  rows_v.at[slot], gsem.at[slot]).start()

        def gat_wait(slot):
            pltpu.make_async_copy(hw_hbm.at[gidx_r.at[slot]],
                                  rows_v.at[slot], gsem.at[slot]).wait()

        def sca_start(slot):
            pltpu.async_copy(rows_v.at[slot], acc_sh.at[dst_r.at[slot]],
                             ssem.at[slot], add=True)

        def sca_wait(slot):
            pltpu.make_async_copy(rows_v.at[slot], acc_sh.at[dst_r.at[slot]],
                                  ssem.at[slot]).wait()

        def steady(t, toff, first=False, last=False):
            if not first or t >= NSLOT - ILAG:
                sca_wait((t - (NSLOT - ILAG)) % NSLOT)
            if not last or t + ILAG < nchunk:
                idx_start((t + ILAG) % NSLOT, toff + ILAG)
            if not last or t + GLAG < nchunk:
                idx_wait((t + GLAG) % NSLOT)
                gat_start((t + GLAG) % NSLOT)
            gat_wait(t % NSLOT)
            sca_start(t % NSLOT)

        for c in range(ILAG):
            idx_start(c % NSLOT, c)
        for c in range(GLAG):
            idx_wait(c % NSLOT)
            gat_start(c % NSLOT)
        zcp.wait()
        plsc.subcore_barrier()

        for b in range(NSLOT):
            steady(b, b, first=True)

        def outer(G, carry):
            for b in range(NSLOT):
                steady(NSLOT + b, G * NSLOT + b)
            return carry

        lax.fori_loop(1, ng - 1, outer, 0)

        for b in range(NSLOT):
            steady((ng - 1) * NSLOT + b, (ng - 1) * NSLOT + b, last=True)

        for c in range(nchunk - (NSLOT - ILAG), nchunk):
            sca_wait(c % NSLOT)

    @pl.when(cid == 0)
    def _():
        make_pipeline(C0, sid * (C0 + C1))

    if C1 > 0:
        @pl.when(cid == 1)
        def _():
            make_pipeline(C1, sid * (C0 + C1) + C0)
    else:
        @pl.when(cid == 1)
        def _():
            zcp.wait()
            plsc.subcore_barrier()

    plsc.subcore_barrier()

    pltpu.sync_copy(acc_sh.at[pl.ds(sid * ROWS_PER_TILE, ROWS_PER_TILE)],
                    out_hbm.at[cid, pl.ds(sid * ROWS_PER_TILE, ROWS_PER_TILE)])


def _edge_aggregate(hw_flat, gidx, dst, zeros):
    mesh = plsc.VectorSubcoreMesh(core_axis_name="c", subcore_axis_name="s")
    return pl.kernel(
        _edge_kernel,
        mesh=mesh,
        out_type=jax.ShapeDtypeStruct((2, NP, H), jnp.float32),
        scratch_types=[
            pltpu.VMEM((NSLOT, CH), jnp.int32),
            pltpu.VMEM((NSLOT, CH), jnp.int32),
            pltpu.VMEM((NSLOT, CH, H), jnp.float32),
            pltpu.VMEM_SHARED((NP, H), jnp.float32),
            pltpu.SemaphoreType.DMA((NSLOT,)),
            pltpu.SemaphoreType.DMA((NSLOT,)),
            pltpu.SemaphoreType.DMA((NSLOT,)),
            pltpu.SemaphoreType.DMA((NSLOT,)),
            pltpu.SemaphoreType.DMA,
        ],
    )(hw_flat, gidx, dst, zeros)


def kernel(x, edge_index, edge_type, W1, Ws1, b1, W2, Ws2, b2, W3, Ws3, b3):
    src = edge_index[0]
    dst = edge_index[1]

    gidx = edge_type * N + src
    npad = EPAD - E
    gidx_p = jnp.concatenate([gidx, jnp.zeros((npad,), jnp.int32)])
    dummy = N + jnp.arange(npad, dtype=jnp.int32) % (NP - N)
    dst_p = jnp.concatenate([dst, dummy])
    zeros = jnp.zeros((NP, H), jnp.float32)

    wc1 = jnp.concatenate([W1, Ws1[None]], axis=0)
    wc2 = jnp.concatenate([W2, Ws2[None]], axis=0)
    wc3 = jnp.concatenate([W3, Ws3[None]], axis=0)

    hw1 = _layer1_matmul(x, wc1)
    agg1 = _edge_aggregate(hw1.reshape(9 * N, H), gidx_p, dst_p, zeros)

    hw2, h1 = _layer_combine_matmul(agg1, hw1, b1.reshape(1, H), wc2)
    agg2 = _edge_aggregate(hw2.reshape(9 * N, H), gidx_p, dst_p, zeros)

    hw3, h2 = _layer_combine_matmul(agg2, hw2, b2.reshape(1, H), wc3)
    agg3 = _edge_aggregate(hw3.reshape(9 * N, H), gidx_p, dst_p, zeros)

    out = _final_pool(h1, h2, agg3, hw3, b3.reshape(1, H))
    return out.reshape(1, 1, 3 * H)


# speedup vs baseline: 4.2215x; 3.6225x over previous
import jax
import jax.numpy as jnp
from jax import lax
from jax.experimental import pallas as pl
from jax.experimental.pallas import tpu as pltpu
from jax.experimental.pallas import tpu_sc as plsc

N = 10000
E = 320000
H = 128

NP = 10112
ROWS_PER_TILE = NP // 16

EPAD = 327680
CH = 64
C0 = 160
C1 = 160

NSLOT = 5
ILAG = 3
GLAG = 2

BN = 400
NB = N // BN


def _mm_body(x_ref, w_ref, o_ref):
    xb = x_ref[...]
    for r in range(9):
        o_ref[r] = jnp.dot(xb, w_ref[r], preferred_element_type=jnp.float32)


def _layer1_matmul(x, wc):
    return pl.pallas_call(
        _mm_body,
        grid=(NB,),
        in_specs=[
            pl.BlockSpec((BN, H), lambda i: (i, 0)),
            pl.BlockSpec((9, H, H), lambda i: (0, 0, 0)),
        ],
        out_specs=pl.BlockSpec((9, BN, H), lambda i: (0, i, 0)),
        out_shape=jax.ShapeDtypeStruct((9, N, H), jnp.float32),
    )(x, wc)


def _mmc_body(a_ref, s_ref, b_ref, w_ref, o_ref, h_ref):
    hb = jnp.maximum(a_ref[0] + a_ref[1] + s_ref[0] + b_ref[...], 0.0)
    h_ref[...] = hb
    for r in range(9):
        o_ref[r] = jnp.dot(hb, w_ref[r], preferred_element_type=jnp.float32)


def _layer_combine_matmul(agg, hw_prev, b, wc):
    return pl.pallas_call(
        _mmc_body,
        grid=(NB,),
        in_specs=[
            pl.BlockSpec((2, BN, H), lambda i: (0, i, 0)),
            pl.BlockSpec((1, BN, H), lambda i: (8, i, 0)),
            pl.BlockSpec((1, H), lambda i: (0, 0)),
            pl.BlockSpec((9, H, H), lambda i: (0, 0, 0)),
        ],
        out_specs=[
            pl.BlockSpec((9, BN, H), lambda i: (0, i, 0)),
            pl.BlockSpec((BN, H), lambda i: (i, 0)),
        ],
        out_shape=[
            jax.ShapeDtypeStruct((9, N, H), jnp.float32),
            jax.ShapeDtypeStruct((N, H), jnp.float32),
        ],
    )(agg, hw_prev, b, wc)


def _final_body(h1_ref, h2_ref, a_ref, s_ref, b_ref, o_ref):
    h3 = jnp.maximum(a_ref[0] + a_ref[1] + s_ref[0] + b_ref[...], 0.0)
    row = jnp.concatenate(
        [
            jnp.sum(h1_ref[...], axis=0, keepdims=True),
            jnp.sum(h2_ref[...], axis=0, keepdims=True),
            jnp.sum(h3, axis=0, keepdims=True),
        ],
        axis=1,
    )
    i = pl.program_id(0)

    @pl.when(i == 0)
    def _():
        o_ref[...] = row

    @pl.when(i != 0)
    def _():
        o_ref[...] += row


def _final_pool(h1, h2, agg, hw_prev, b):
    return pl.pallas_call(
        _final_body,
        grid=(NB,),
        in_specs=[
            pl.BlockSpec((BN, H), lambda i: (i, 0)),
            pl.BlockSpec((BN, H), lambda i: (i, 0)),
            pl.BlockSpec((2, BN, H), lambda i: (0, i, 0)),
            pl.BlockSpec((1, BN, H), lambda i: (8, i, 0)),
            pl.BlockSpec((1, H), lambda i: (0, 0)),
        ],
        out_specs=pl.BlockSpec((1, 3 * H), lambda i: (0, 0)),
        out_shape=jax.ShapeDtypeStruct((1, 3 * H), jnp.float32),
    )(h1, h2, agg, hw_prev, b)


def _edge_kernel(hw_hbm, gidx_hbm, dst_hbm, zeros_hbm, out_hbm,
                 gidx_r, dst_r, rows_v, acc_sh, isem, jsem, gsem, ssem, zsem):
    cid = lax.axis_index("c")
    sid = lax.axis_index("s")

    zcp = pltpu.make_async_copy(
        zeros_hbm.at[pl.ds(sid * ROWS_PER_TILE, ROWS_PER_TILE)],
        acc_sh.at[pl.ds(sid * ROWS_PER_TILE, ROWS_PER_TILE)], zsem)
    zcp.start()

    def make_pipeline(nchunk, cbase):
        ng = nchunk // NSLOT

        def idx_start(slot, c):
            base = (cbase + c) * CH
            pltpu.make_async_copy(gidx_hbm.at[pl.ds(base, CH)],
                                  gidx_r.at[slot], isem.at[slot]).start()
            pltpu.make_async_copy(dst_hbm.at[pl.ds(base, CH)],
                                  dst_r.at[slot], jsem.at[slot]).start()

        def idx_wait(slot):
            pltpu.make_async_copy(gidx_hbm.at[pl.ds(0, CH)],
                                  gidx_r.at[slot], isem.at[slot]).wait()
            pltpu.make_async_copy(dst_hbm.at[pl.ds(0, CH)],
                                  dst_r.at[slot], jsem.at[slot]).wait()

        def gat_start(slot):
            pltpu.make_async_copy(hw_hbm.at[gidx_r.at[slot]],
                                  rows_v.at[slot], gsem.at[slot]).start()

        def gat_wait(slot):
            pltpu.make_async_copy(hw_hbm.at[gidx_r.at[slot]],
                                  rows_v.at[slot], gsem.at[slot]).wait()

        def sca_start(slot):
            pltpu.async_copy(rows_v.at[slot], acc_sh.at[dst_r.at[slot]],
                             ssem.at[slot], add=True)

        def sca_wait(slot):
            pltpu.make_async_copy(rows_v.at[slot], acc_sh.at[dst_r.at[slot]],
                                  ssem.at[slot]).wait()

        def steady(t, toff, first=False, last=False):
            if not first or t >= NSLOT - ILAG:
                sca_wait((t - (NSLOT - ILAG)) % NSLOT)
            if not last or t + ILAG < nchunk:
                idx_start((t + ILAG) % NSLOT, toff + ILAG)
            if not last or t + GLAG < nchunk:
                idx_wait((t + GLAG) % NSLOT)
                gat_start((t + GLAG) % NSLOT)
            gat_wait(t % NSLOT)
            sca_start(t % NSLOT)

        for c in range(ILAG):
            idx_start(c % NSLOT, c)
        for c in range(GLAG):
            idx_wait(c % NSLOT)
            gat_start(c % NSLOT)
        zcp.wait()
        plsc.subcore_barrier()

        for b in range(NSLOT):
            steady(b, b, first=True)

        def outer(G, carry):
            for b in range(NSLOT):
                steady(NSLOT + b, G * NSLOT + b)
            return carry

        lax.fori_loop(1, ng - 1, outer, 0)

        for b in range(NSLOT):
            steady((ng - 1) * NSLOT + b, (ng - 1) * NSLOT + b, last=True)

        for c in range(nchunk - (NSLOT - ILAG), nchunk):
            sca_wait(c % NSLOT)

    @pl.when(cid == 0)
    def _():
        make_pipeline(C0, sid * (C0 + C1))

    if C1 > 0:
        @pl.when(cid == 1)
        def _():
            make_pipeline(C1, sid * (C0 + C1) + C0)
    else:
        @pl.when(cid == 1)
        def _():
            zcp.wait()
            plsc.subcore_barrier()

    plsc.subcore_barrier()

    pltpu.sync_copy(acc_sh.at[pl.ds(sid * ROWS_PER_TILE, ROWS_PER_TILE)],
                    out_hbm.at[cid, pl.ds(sid * ROWS_PER_TILE, ROWS_PER_TILE)])


def _edge_aggregate(hw_flat, gidx, dst, zeros):
    mesh = plsc.VectorSubcoreMesh(core_axis_name="c", subcore_axis_name="s")
    return pl.kernel(
        _edge_kernel,
        mesh=mesh,
        out_type=jax.ShapeDtypeStruct((2, NP, H), jnp.float32),
        scratch_types=[
            pltpu.VMEM((NSLOT, CH), jnp.int32),
            pltpu.VMEM((NSLOT, CH), jnp.int32),
            pltpu.VMEM((NSLOT, CH, H), jnp.float32),
            pltpu.VMEM_SHARED((NP, H), jnp.float32),
            pltpu.SemaphoreType.DMA((NSLOT,)),
            pltpu.SemaphoreType.DMA((NSLOT,)),
            pltpu.SemaphoreType.DMA((NSLOT,)),
            pltpu.SemaphoreType.DMA((NSLOT,)),
            pltpu.SemaphoreType.DMA,
        ],
    )(hw_flat, gidx, dst, zeros)


def kernel(x, edge_index, edge_type, W1, Ws1, b1, W2, Ws2, b2, W3, Ws3, b3):
    src = edge_index[0]
    dst = edge_index[1]

    gidx = edge_type * N + src
    npad = EPAD - E
    pad_ar = jnp.arange(npad, dtype=jnp.int32)
    gidx_p = jnp.concatenate([gidx, pad_ar % N])
    dst_p = jnp.concatenate([dst, N + pad_ar % (NP - N)])
    zeros = jnp.zeros((NP, H), jnp.float32)

    wc1 = jnp.concatenate([W1, Ws1[None]], axis=0)
    wc2 = jnp.concatenate([W2, Ws2[None]], axis=0)
    wc3 = jnp.concatenate([W3, Ws3[None]], axis=0)

    hw1 = _layer1_matmul(x, wc1)
    agg1 = _edge_aggregate(hw1.reshape(9 * N, H), gidx_p, dst_p, zeros)

    hw2, h1 = _layer_combine_matmul(agg1, hw1, b1.reshape(1, H), wc2)
    agg2 = _edge_aggregate(hw2.reshape(9 * N, H), gidx_p, dst_p, zeros)

    hw3, h2 = _layer_combine_matmul(agg2, hw2, b2.reshape(1, H), wc3)
    agg3 = _edge_aggregate(hw3.reshape(9 * N, H), gidx_p, dst_p, zeros)

    out = _final_pool(h1, h2, agg3, hw3, b3.reshape(1, H))
    return out.reshape(1, 1, 3 * H)
